# Initial kernel scaffold; baseline (speedup 1.0000x reference)
import jax
import jax.numpy as jnp
from jax.experimental import pallas as pl


def kernel(X, K, dlnf_grid, radius, para_lut):
    raise NotImplementedError("write your pallas kernel here")


# speedup vs baseline: 2.2503x; 2.2503x over previous
import jax
import jax.numpy as jnp
from jax.experimental import pallas as pl

_KC = 16


def _peak_kernel(x_ref, g_ref, lut_ref, sel_out, val_out, fr_out, dr_out):
    G, D, F = x_ref.shape
    LN = lut_ref.shape[1]
    amp = jnp.abs(x_ref[...])

    fpad = jnp.full((G, D, 3), -1.0, jnp.float32)
    xf = jnp.concatenate([fpad, amp, fpad], axis=2)
    pf = xf[:, :, 0:F]
    for k in range(1, 7):
        pf = jnp.maximum(pf, xf[:, :, k:k + F])
    dpad = jnp.full((G, 3, F), -1.0, jnp.float32)
    xd = jnp.concatenate([dpad, pf, dpad], axis=1)
    pooled = xd[:, 0:D, :]
    for k in range(1, 7):
        pooled = jnp.maximum(pooled, xd[:, k:k + D, :])

    colmax = jnp.max(amp, axis=1)
    dio = jax.lax.broadcasted_iota(jnp.int32, (G, D, F), 1)
    ismax = amp == colmax[:, None, :]
    bestd = jnp.min(jnp.where(ismax, dio, D), axis=1)
    E = dio == bestd[:, None, :]

    pr = jnp.max(jnp.where(E, pooled, -1.0), axis=1)
    v = jnp.where(colmax >= pr, colmax, 0.0)

    a_m = jnp.concatenate([amp[:, :, 0:1], amp[:, :, 0:F - 2],
                           amp[:, :, F - 3:F - 2]], axis=2)
    a_0 = jnp.concatenate([amp[:, :, 1:2], amp[:, :, 1:F - 1],
                           amp[:, :, F - 2:F - 1]], axis=2)
    a_p = jnp.concatenate([amp[:, :, 2:3], amp[:, :, 2:F],
                           amp[:, :, F - 1:F]], axis=2)
    yfm = jnp.sum(jnp.where(E, a_m, 0.0), axis=1)
    yf0 = jnp.sum(jnp.where(E, a_0, 0.0), axis=1)
    yfp = jnp.sum(jnp.where(E, a_p, 0.0), axis=1)

    di = jnp.clip(bestd, 1, D - 2)
    E2 = dio == di[:, None, :]
    a_up = jnp.concatenate([amp[:, 0:1, :], amp[:, 0:D - 1, :]], axis=1)
    a_dn = jnp.concatenate([amp[:, 1:D, :], amp[:, D - 1:D, :]], axis=1)
    ydm = jnp.sum(jnp.where(E2, a_up, 0.0), axis=1)
    yd0 = jnp.sum(jnp.where(E2, amp, 0.0), axis=1)
    ydp = jnp.sum(jnp.where(E2, a_dn, 0.0), axis=1)

    fio = jax.lax.broadcasted_iota(jnp.int32, (G, F), 1)
    flat = bestd * F + fio
    big = D * F
    work = v
    vals, sels = [], []
    for _ in range(_KC):
        m = jnp.max(work, axis=1, keepdims=True)
        s = jnp.min(jnp.where(work == m, flat, big), axis=1, keepdims=True)
        vals.append(m)
        sels.append(s)
        work = jnp.where(flat == s, -1.0, work)
    val16 = jnp.concatenate(vals, axis=1)
    sel16 = jnp.concatenate(sels, axis=1)

    ohb = sel16[:, :, None] == flat[:, None, :]

    def gsel_i(arr):
        return jnp.sum(jnp.where(ohb, arr[:, None, :], 0), axis=2)

    def gsel_f(arr):
        return jnp.sum(jnp.where(ohb, arr[:, None, :], 0.0), axis=2)

    f16 = gsel_i(fio)
    d16 = gsel_i(bestd)
    yfm16 = gsel_f(yfm)
    yf016 = gsel_f(yf0)
    yfp16 = gsel_f(yfp)
    ydm16 = gsel_f(ydm)
    yd016 = gsel_f(yd0)
    ydp16 = gsel_f(ydp)

    f_denom = yfm16 - 2.0 * yf016 + yfp16
    f_bad = jnp.abs(f_denom) < 1e-12
    f_safe = jnp.where(f_bad, 1.0, f_denom)
    f_delta = jnp.where(f_bad, 0.0, 0.5 * (yfm16 - yfp16) / f_safe)
    f_delta = jnp.clip(f_delta, -0.5, 0.5)
    sign = jnp.sign(f_delta)
    mag = jnp.abs(f_delta)
    pos = mag / 0.5 * (LN - 1)
    i0 = jnp.clip(jnp.floor(pos).astype(jnp.int32), 0, LN - 2)
    frac = pos - i0.astype(jnp.float32)
    li = jax.lax.broadcasted_iota(jnp.int32, (G, _KC, LN), 2)
    lut3 = lut_ref[...].reshape(1, 1, LN)
    l0 = jnp.sum(jnp.where(li == i0[:, :, None], lut3, 0.0), axis=2)
    l1 = jnp.sum(jnp.where(li == (i0 + 1)[:, :, None], lut3, 0.0), axis=2)
    f_delta_c = sign * (l0 * (1.0 - frac) + l1 * frac)
    fi16 = jnp.clip(f16, 1, F - 2)
    fr_out[...] = fi16.astype(jnp.float32) + f_delta_c

    d_denom = ydm16 - 2.0 * yd016 + ydp16
    d_bad = jnp.abs(d_denom) < 1e-12
    d_safe = jnp.where(d_bad, 1.0, d_denom)
    d_delta = jnp.where(d_bad, 0.0, 0.5 * (ydm16 - ydp16) / d_safe)
    d_delta = jnp.clip(d_delta, -0.5, 0.5)
    step = g_ref[0, 1] - g_ref[0, 0]
    di16 = jnp.clip(d16, 1, D - 2)
    gi = jax.lax.broadcasted_iota(jnp.int32, (G, _KC, D), 2)
    g3 = g_ref[...].reshape(1, 1, D)
    gval = jnp.sum(jnp.where(gi == di16[:, :, None], g3, 0.0), axis=2)
    dr_out[...] = gval + d_delta * step

    sel_out[...] = sel16
    val_out[...] = val16


def kernel(X, K, dlnf_grid, radius, para_lut):
    B, W, D, Fk = X.shape
    BW = B * W
    G = 8
    Xr = X.reshape(BW, D, Fk)
    g2 = dlnf_grid.reshape(1, D)
    lut2 = para_lut.reshape(1, -1)
    lutn = lut2.shape[1]
    sel, vals, fr, dr = pl.pallas_call(
        _peak_kernel,
        grid=(BW // G,),
        in_specs=[
            pl.BlockSpec((G, D, Fk), lambda i: (i, 0, 0)),
            pl.BlockSpec((1, D), lambda i: (0, 0)),
            pl.BlockSpec((1, lutn), lambda i: (0, 0)),
        ],
        out_specs=[
            pl.BlockSpec((G, _KC), lambda i: (i, 0)),
            pl.BlockSpec((G, _KC), lambda i: (i, 0)),
            pl.BlockSpec((G, _KC), lambda i: (i, 0)),
            pl.BlockSpec((G, _KC), lambda i: (i, 0)),
        ],
        out_shape=[
            jax.ShapeDtypeStruct((BW, _KC), jnp.int32),
            jax.ShapeDtypeStruct((BW, _KC), jnp.float32),
            jax.ShapeDtypeStruct((BW, _KC), jnp.float32),
            jax.ShapeDtypeStruct((BW, _KC), jnp.float32),
        ],
    )(Xr, g2, lut2)
    offset = (jnp.asarray(K) - 16 + jnp.asarray(radius) - 3).astype(jnp.int32)
    flat2 = sel + offset
    d_idx = flat2 // Fk
    f_idx = flat2 % Fk
    peaks = jnp.stack([d_idx, f_idx], axis=-1).reshape(B, W, _KC, 2)
    return (peaks,
            fr.reshape(B, W, _KC),
            dr.reshape(B, W, _KC),
            vals.reshape(B, W, _KC))


# speedup vs baseline: 3.2744x; 1.4551x over previous
import jax
import jax.numpy as jnp
from jax.experimental import pallas as pl

_KC = 16


def _peak_kernel(x_ref, g_ref, lut_ref, sel_out, val_out, fr_out, dr_out):
    G, D, F = x_ref.shape
    LN = lut_ref.shape[1]
    amp = jnp.abs(x_ref[...])

    dpad = jnp.full((G, 3, F), -1.0, jnp.float32)
    xd = jnp.concatenate([dpad, amp, dpad], axis=1)
    m2d = jnp.maximum(xd[:, 0:D + 5, :], xd[:, 1:D + 6, :])
    m4d = jnp.maximum(m2d[:, 0:D + 3, :], m2d[:, 2:D + 5, :])
    rowpool = jnp.maximum(jnp.maximum(m4d[:, 0:D, :], m2d[:, 4:D + 4, :]),
                          xd[:, 6:D + 6, :])
    fpad = jnp.full((G, D, 3), -1.0, jnp.float32)
    xf = jnp.concatenate([fpad, rowpool, fpad], axis=2)
    m2f = jnp.maximum(xf[:, :, 0:F + 5], xf[:, :, 1:F + 6])
    m4f = jnp.maximum(m2f[:, :, 0:F + 3], m2f[:, :, 2:F + 5])
    pooled = jnp.maximum(jnp.maximum(m4f[:, :, 0:F], m2f[:, :, 4:F + 4]),
                         xf[:, :, 6:F + 6])

    colmax = jnp.max(amp, axis=1)
    dio = jax.lax.broadcasted_iota(jnp.int32, (G, D, F), 1)
    ismax = amp == colmax[:, None, :]
    bestd = jnp.min(jnp.where(ismax, dio, D), axis=1)
    E = dio == bestd[:, None, :]

    pr = jnp.max(jnp.where(E, pooled, -1.0), axis=1)
    v = jnp.where(colmax >= pr, colmax, 0.0)

    fio = jax.lax.broadcasted_iota(jnp.int32, (G, F), 1)
    flat = bestd * F + fio
    big = D * F
    work = v
    vals, sels = [], []
    for _ in range(_KC):
        m = jnp.max(work, axis=1, keepdims=True)
        s = jnp.min(jnp.where(work == m, flat, big), axis=1, keepdims=True)
        vals.append(m)
        sels.append(s)
        work = jnp.where(flat == s, -1.0, work)
    val16 = jnp.concatenate(vals, axis=1)
    sel16 = jnp.concatenate(sels, axis=1)

    ohb = sel16[:, :, None] == flat[:, None, :]
    fio3 = jax.lax.broadcasted_iota(jnp.int32, (G, _KC, F), 2)
    f16 = jnp.sum(jnp.where(ohb, fio3, 0), axis=2)
    d16 = jnp.sum(jnp.where(ohb, bestd[:, None, :], 0), axis=2)
    fi16 = jnp.clip(f16, 1, F - 2)
    di16 = jnp.clip(d16, 1, D - 2)

    ohfm = (fio3 == (fi16 - 1)[:, :, None]).astype(jnp.float32)
    ohf0 = (fio3 == fi16[:, :, None]).astype(jnp.float32)
    ohfp = (fio3 == (fi16 + 1)[:, :, None]).astype(jnp.float32)
    ohfj = (fio3 == f16[:, :, None]).astype(jnp.float32)
    oh = jnp.concatenate([ohfm, ohf0, ohfp, ohfj], axis=1)
    cols = []
    for g in range(G):
        cols.append(jax.lax.dot_general(
            oh[g], amp[g], (((1,), (1,)), ((), ())),
            precision=jax.lax.Precision.HIGHEST,
            preferred_element_type=jnp.float32))
    colsq = jnp.stack(cols, axis=0)

    dio16 = jax.lax.broadcasted_iota(jnp.int32, (G, _KC, D), 2)
    ed0 = dio16 == d16[:, :, None]
    edc = dio16 == di16[:, :, None]
    edm = dio16 == (di16 - 1)[:, :, None]
    edp = dio16 == (di16 + 1)[:, :, None]
    yfm16 = jnp.sum(jnp.where(ed0, colsq[:, 0:16, :], 0.0), axis=2)
    yf016 = jnp.sum(jnp.where(ed0, colsq[:, 16:32, :], 0.0), axis=2)
    yfp16 = jnp.sum(jnp.where(ed0, colsq[:, 32:48, :], 0.0), axis=2)
    ydm16 = jnp.sum(jnp.where(edm, colsq[:, 48:64, :], 0.0), axis=2)
    yd016 = jnp.sum(jnp.where(edc, colsq[:, 48:64, :], 0.0), axis=2)
    ydp16 = jnp.sum(jnp.where(edp, colsq[:, 48:64, :], 0.0), axis=2)

    f_denom = yfm16 - 2.0 * yf016 + yfp16
    f_bad = jnp.abs(f_denom) < 1e-12
    f_safe = jnp.where(f_bad, 1.0, f_denom)
    f_delta = jnp.where(f_bad, 0.0, 0.5 * (yfm16 - yfp16) / f_safe)
    f_delta = jnp.clip(f_delta, -0.5, 0.5)
    sign = jnp.sign(f_delta)
    mag = jnp.abs(f_delta)
    pos = mag / 0.5 * (LN - 1)
    i0 = jnp.clip(jnp.floor(pos).astype(jnp.int32), 0, LN - 2)
    frac = pos - i0.astype(jnp.float32)
    li = jax.lax.broadcasted_iota(jnp.int32, (G, _KC, LN), 2)
    lut3 = lut_ref[...].reshape(1, 1, LN)
    l0 = jnp.sum(jnp.where(li == i0[:, :, None], lut3, 0.0), axis=2)
    l1 = jnp.sum(jnp.where(li == (i0 + 1)[:, :, None], lut3, 0.0), axis=2)
    f_delta_c = sign * (l0 * (1.0 - frac) + l1 * frac)
    fr_out[...] = fi16.astype(jnp.float32) + f_delta_c

    d_denom = ydm16 - 2.0 * yd016 + ydp16
    d_bad = jnp.abs(d_denom) < 1e-12
    d_safe = jnp.where(d_bad, 1.0, d_denom)
    d_delta = jnp.where(d_bad, 0.0, 0.5 * (ydm16 - ydp16) / d_safe)
    d_delta = jnp.clip(d_delta, -0.5, 0.5)
    step = g_ref[0, 1] - g_ref[0, 0]
    gi = jax.lax.broadcasted_iota(jnp.int32, (G, _KC, D), 2)
    g3 = g_ref[...].reshape(1, 1, D)
    gval = jnp.sum(jnp.where(gi == di16[:, :, None], g3, 0.0), axis=2)
    dr_out[...] = gval + d_delta * step

    sel_out[...] = sel16
    val_out[...] = val16


def kernel(X, K, dlnf_grid, radius, para_lut):
    B, W, D, Fk = X.shape
    BW = B * W
    G = 32
    Xr = X.reshape(BW, D, Fk)
    g2 = dlnf_grid.reshape(1, D)
    lut2 = para_lut.reshape(1, -1)
    lutn = lut2.shape[1]
    sel, vals, fr, dr = pl.pallas_call(
        _peak_kernel,
        grid=(BW // G,),
        in_specs=[
            pl.BlockSpec((G, D, Fk), lambda i: (i, 0, 0)),
            pl.BlockSpec((1, D), lambda i: (0, 0)),
            pl.BlockSpec((1, lutn), lambda i: (0, 0)),
        ],
        out_specs=[
            pl.BlockSpec((G, _KC), lambda i: (i, 0)),
            pl.BlockSpec((G, _KC), lambda i: (i, 0)),
            pl.BlockSpec((G, _KC), lambda i: (i, 0)),
            pl.BlockSpec((G, _KC), lambda i: (i, 0)),
        ],
        out_shape=[
            jax.ShapeDtypeStruct((BW, _KC), jnp.int32),
            jax.ShapeDtypeStruct((BW, _KC), jnp.float32),
            jax.ShapeDtypeStruct((BW, _KC), jnp.float32),
            jax.ShapeDtypeStruct((BW, _KC), jnp.float32),
        ],
    )(Xr, g2, lut2)
    offset = (jnp.asarray(K) - 16 + jnp.asarray(radius) - 3).astype(jnp.int32)
    flat2 = sel + offset
    d_idx = flat2 // Fk
    f_idx = flat2 % Fk
    peaks = jnp.stack([d_idx, f_idx], axis=-1).reshape(B, W, _KC, 2)
    return (peaks,
            fr.reshape(B, W, _KC),
            dr.reshape(B, W, _KC),
            vals.reshape(B, W, _KC))


# speedup vs baseline: 4.2035x; 1.2837x over previous
import functools

import jax
import jax.numpy as jnp
from jax import lax
from jax.experimental import pallas as pl
from jax.experimental.pallas import tpu as pltpu
from jax.experimental.pallas import tpu_sc as plsc

_KC = 16
_FP = 528


def _stage1_kernel(x_ref, v_out, flat_out):
    G, D, F = x_ref.shape
    amp = jnp.abs(x_ref[...])

    dpad = jnp.full((G, 3, F), -1.0, jnp.float32)
    xd = jnp.concatenate([dpad, amp, dpad], axis=1)
    m2d = jnp.maximum(xd[:, 0:D + 5, :], xd[:, 1:D + 6, :])
    m4d = jnp.maximum(m2d[:, 0:D + 3, :], m2d[:, 2:D + 5, :])
    rowpool = jnp.maximum(jnp.maximum(m4d[:, 0:D, :], m2d[:, 4:D + 4, :]),
                          xd[:, 6:D + 6, :])
    fpad = jnp.full((G, D, 3), -1.0, jnp.float32)
    xf = jnp.concatenate([fpad, rowpool, fpad], axis=2)
    m2f = jnp.maximum(xf[:, :, 0:F + 5], xf[:, :, 1:F + 6])
    m4f = jnp.maximum(m2f[:, :, 0:F + 3], m2f[:, :, 2:F + 5])
    pooled = jnp.maximum(jnp.maximum(m4f[:, :, 0:F], m2f[:, :, 4:F + 4]),
                         xf[:, :, 6:F + 6])

    colmax = jnp.max(amp, axis=1)
    dio = jax.lax.broadcasted_iota(jnp.int32, (G, D, F), 1)
    ismax = amp == colmax[:, None, :]
    bestd = jnp.min(jnp.where(ismax, dio, D), axis=1)
    E = dio == bestd[:, None, :]
    pr = jnp.max(jnp.where(E, pooled, -1.0), axis=1)
    v = jnp.where(colmax >= pr, colmax, 0.0)

    fio = jax.lax.broadcasted_iota(jnp.int32, (G, F), 1)
    flat = bestd * F + fio

    npad = _FP - F
    v_out[...] = jnp.concatenate(
        [v, jnp.full((G, npad), -1.0, jnp.float32)], axis=1)
    flat_out[...] = jnp.concatenate(
        [flat, jnp.zeros((G, npad), jnp.int32)], axis=1)


def _stage1(Xr, G):
    BW, D, Fk = Xr.shape
    fspec = pl.BlockSpec((G, _FP), lambda i: (i, 0))
    return pl.pallas_call(
        _stage1_kernel,
        grid=(BW // G,),
        in_specs=[pl.BlockSpec((G, D, Fk), lambda i: (i, 0, 0))],
        out_specs=[fspec, fspec],
        out_shape=[jax.ShapeDtypeStruct((BW, _FP), jnp.float32),
                   jax.ShapeDtypeStruct((BW, _FP), jnp.int32)],
    )(Xr)


def _make_stage2(BW, D, Fk, LN):
    NW = 32
    WPW = BW // NW
    NV = _FP // 16
    plane = D * Fk
    mesh = plsc.VectorSubcoreMesh(core_axis_name="c", subcore_axis_name="s")
    f_out = jax.ShapeDtypeStruct((BW, _KC), jnp.float32)
    i_out = jax.ShapeDtypeStruct((BW, _KC), jnp.int32)

    @functools.partial(
        pl.kernel, mesh=mesh,
        out_type=[i_out, f_out, f_out, f_out],
        compiler_params=pltpu.CompilerParams(needs_layout_passes=False),
        scratch_types=[
            pltpu.VMEM((WPW, _FP), jnp.float32),
            pltpu.VMEM((WPW, _FP), jnp.int32),
            pltpu.VMEM((LN,), jnp.float32),
            pltpu.VMEM((D,), jnp.float32),
            pltpu.VMEM((96,), jnp.int32),
            pltpu.VMEM((96,), jnp.float32),
            pltpu.VMEM((16,), jnp.float32),
            pltpu.VMEM((16,), jnp.float32),
            pltpu.VMEM((WPW, _KC), jnp.int32),
            pltpu.VMEM((WPW, _KC), jnp.float32),
            pltpu.VMEM((WPW, _KC), jnp.float32),
            pltpu.VMEM((WPW, _KC), jnp.float32),
            pltpu.SemaphoreType.DMA,
        ],
    )
    def sc_kernel(v_h, flat_h, x_h, lut_h, grid_h, step_h,
                  sel_o, val_o, fr_o, dr_o,
                  v_s, flat_s, lut_s, grid_s, idx_s, gat_s, bk_s, step_s,
                  sel_s, val_s, fr_s, dr_s, sem):
        wid = lax.axis_index("s") * 2 + lax.axis_index("c")
        base = wid * WPW
        rows = pl.ds(base, WPW)
        pltpu.sync_copy(v_h.at[rows, :], v_s)
        pltpu.sync_copy(flat_h.at[rows, :], flat_s)
        pltpu.sync_copy(lut_h, lut_s)
        pltpu.sync_copy(grid_h, grid_s)
        pltpu.sync_copy(step_h, step_s)

        zeros16 = jnp.zeros((16,), jnp.int32)
        step = step_s[...]

        def window_body(wl, carry):
            init_k = jnp.full((16,), -2.0, jnp.float32)
            init_i = zeros16

            def merge_body(r, kc):
                bk, bi = kc
                kv = v_s[wl, pl.ds(r * 16, 16)]
                fv = flat_s[wl, pl.ds(r * 16, 16)]
                kv2, fv2 = plsc.sort_key_val(kv, fv, descending=True)
                rk = lax.rev(bk, (0,))
                ri = lax.rev(bi, (0,))
                m = kv2 >= rk
                nk = jnp.where(m, kv2, rk)
                ni = jnp.where(m, fv2, ri)
                sk, si = plsc.sort_key_val(nk, ni, descending=True)
                return (sk, si)

            best_k, _ = lax.fori_loop(
                0, NV, merge_body, (init_k, init_i))

            bk_s[...] = best_k
            lane = lax.iota(jnp.int32, 16)
            big = jnp.full((16,), D * Fk, jnp.int32)

            def assign_body(j, kc):
                bi, prev_v, prev_f = kc
                jv = zeros16 + j
                bkj = plsc.load_gather(bk_s, [jv])
                thresh = jnp.where(bkj == prev_v, prev_f, zeros16 - 1)

                def scan_body(r, acc):
                    kv = v_s[wl, pl.ds(r * 16, 16)]
                    fv = flat_s[wl, pl.ds(r * 16, 16)]
                    hit = (kv == bkj) & (fv > thresh)
                    return jnp.minimum(acc, jnp.where(hit, fv, big))

                part = lax.fori_loop(0, NV, scan_body, big)
                mj = jnp.min(part, axis=0)
                mjs = zeros16 + mj
                bi = jnp.where(lane == jv, mjs, bi)
                return (bi, bkj, mjs)

            best_i, _, _ = lax.fori_loop(
                0, _KC, assign_body,
                (zeros16, jnp.full((16,), -3.0, jnp.float32), zeros16 - 1))

            f16 = best_i % Fk
            d16 = best_i // Fk
            fi16 = jnp.clip(f16, 1, Fk - 2)
            di16 = jnp.clip(d16, 1, D - 2)
            xbase = (base + wl) * plane
            rowb = xbase + d16 * Fk
            idx_s[pl.ds(0, 16)] = rowb + (fi16 - 1)
            idx_s[pl.ds(16, 16)] = rowb + fi16
            idx_s[pl.ds(32, 16)] = rowb + (fi16 + 1)
            colb = xbase + f16
            idx_s[pl.ds(48, 16)] = colb + (di16 - 1) * Fk
            idx_s[pl.ds(64, 16)] = colb + di16 * Fk
            idx_s[pl.ds(80, 16)] = colb + (di16 + 1) * Fk
            pltpu.async_copy(x_h.at[idx_s], gat_s, sem).wait()

            yfm16 = jnp.abs(gat_s[pl.ds(0, 16)])
            yf016 = jnp.abs(gat_s[pl.ds(16, 16)])
            yfp16 = jnp.abs(gat_s[pl.ds(32, 16)])
            ydm16 = jnp.abs(gat_s[pl.ds(48, 16)])
            yd016 = jnp.abs(gat_s[pl.ds(64, 16)])
            ydp16 = jnp.abs(gat_s[pl.ds(80, 16)])

            f_denom = yfm16 - 2.0 * yf016 + yfp16
            f_bad = jnp.abs(f_denom) < 1e-12
            f_safe = jnp.where(f_bad, 1.0, f_denom)
            f_delta = jnp.where(f_bad, 0.0, 0.5 * (yfm16 - yfp16) / f_safe)
            f_delta = jnp.clip(f_delta, -0.5, 0.5)
            sign = jnp.sign(f_delta)
            mag = jnp.abs(f_delta)
            pos = mag / 0.5 * (LN - 1)
            i0 = jnp.clip(pos.astype(jnp.int32), 0, LN - 2)
            frac = pos - i0.astype(jnp.float32)
            l0 = plsc.load_gather(lut_s, [i0])
            l1 = plsc.load_gather(lut_s, [i0 + 1])
            f_delta_c = sign * (l0 * (1.0 - frac) + l1 * frac)
            fr16 = fi16.astype(jnp.float32) + f_delta_c

            d_denom = ydm16 - 2.0 * yd016 + ydp16
            d_bad = jnp.abs(d_denom) < 1e-12
            d_safe = jnp.where(d_bad, 1.0, d_denom)
            d_delta = jnp.where(d_bad, 0.0, 0.5 * (ydm16 - ydp16) / d_safe)
            d_delta = jnp.clip(d_delta, -0.5, 0.5)
            gv = plsc.load_gather(grid_s, [di16])
            dr16 = gv + d_delta * step

            sel_s[wl, :] = best_i
            val_s[wl, :] = best_k
            fr_s[wl, :] = fr16
            dr_s[wl, :] = dr16
            return carry

        lax.fori_loop(0, WPW, window_body, 0)

        pltpu.sync_copy(sel_s, sel_o.at[rows, :])
        pltpu.sync_copy(val_s, val_o.at[rows, :])
        pltpu.sync_copy(fr_s, fr_o.at[rows, :])
        pltpu.sync_copy(dr_s, dr_o.at[rows, :])

    return sc_kernel


def kernel(X, K, dlnf_grid, radius, para_lut):
    B, W, D, Fk = X.shape
    BW = B * W
    G = 32
    Xr = X.reshape(BW, D, Fk)
    v, flat = _stage1(Xr, G)
    LN = para_lut.shape[0]
    sc = _make_stage2(BW, D, Fk, LN)
    step_arr = jnp.broadcast_to(dlnf_grid[1] - dlnf_grid[0], (16,))
    sel, vals, fr, dr = sc(v, flat, X.reshape(-1), para_lut, dlnf_grid,
                           step_arr)
    offset = (jnp.asarray(K) - 16 + jnp.asarray(radius) - 3).astype(jnp.int32)
    flat2 = sel + offset
    d_idx = flat2 // Fk
    f_idx = flat2 % Fk
    peaks = jnp.stack([d_idx, f_idx], axis=-1).reshape(B, W, _KC, 2)
    return (peaks,
            fr.reshape(B, W, _KC),
            dr.reshape(B, W, _KC),
            vals.reshape(B, W, _KC))


# speedup vs baseline: 4.3430x; 1.0332x over previous
import functools

import jax
import jax.numpy as jnp
from jax import lax
from jax.experimental import pallas as pl
from jax.experimental.pallas import tpu as pltpu
from jax.experimental.pallas import tpu_sc as plsc

_KC = 16
_FP = 528


def _stage1_kernel(x_ref, v_out, flat_out):
    G, D, F = x_ref.shape
    amp = jnp.abs(x_ref[...])

    dpad = jnp.full((G, 3, F), -1.0, jnp.float32)
    xd = jnp.concatenate([dpad, amp, dpad], axis=1)
    m2d = jnp.maximum(xd[:, 0:D + 5, :], xd[:, 1:D + 6, :])
    m4d = jnp.maximum(m2d[:, 0:D + 3, :], m2d[:, 2:D + 5, :])
    rowpool = jnp.maximum(jnp.maximum(m4d[:, 0:D, :], m2d[:, 4:D + 4, :]),
                          xd[:, 6:D + 6, :])
    fpad = jnp.full((G, D, 3), -1.0, jnp.float32)
    xf = jnp.concatenate([fpad, rowpool, fpad], axis=2)
    m2f = jnp.maximum(xf[:, :, 0:F + 5], xf[:, :, 1:F + 6])
    m4f = jnp.maximum(m2f[:, :, 0:F + 3], m2f[:, :, 2:F + 5])
    pooled = jnp.maximum(jnp.maximum(m4f[:, :, 0:F], m2f[:, :, 4:F + 4]),
                         xf[:, :, 6:F + 6])

    colmax = jnp.max(amp, axis=1)
    dio = jax.lax.broadcasted_iota(jnp.int32, (G, D, F), 1)
    ismax = amp == colmax[:, None, :]
    bestd = jnp.min(jnp.where(ismax, dio, D), axis=1)
    E = dio == bestd[:, None, :]
    pr = jnp.max(jnp.where(E, pooled, -1.0), axis=1)
    v = jnp.where(colmax >= pr, colmax, 0.0)

    fio = jax.lax.broadcasted_iota(jnp.int32, (G, F), 1)
    flat = bestd * F + fio

    npad = _FP - F
    v_out[...] = jnp.concatenate(
        [v, jnp.full((G, npad), -1.0, jnp.float32)], axis=1)
    flat_out[...] = jnp.concatenate(
        [flat, jnp.zeros((G, npad), jnp.int32)], axis=1)


def _stage1(Xr, G):
    BW, D, Fk = Xr.shape
    fspec = pl.BlockSpec((G, _FP), lambda i: (i, 0))
    return pl.pallas_call(
        _stage1_kernel,
        grid=(BW // G,),
        in_specs=[pl.BlockSpec((G, D, Fk), lambda i: (i, 0, 0))],
        out_specs=[fspec, fspec],
        out_shape=[jax.ShapeDtypeStruct((BW, _FP), jnp.float32),
                   jax.ShapeDtypeStruct((BW, _FP), jnp.int32)],
    )(Xr)


def _make_stage2(BW, D, Fk, LN):
    NW = 32
    WPW = BW // NW
    NV = _FP // 16
    plane = D * Fk
    mesh = plsc.VectorSubcoreMesh(core_axis_name="c", subcore_axis_name="s")
    f_out = jax.ShapeDtypeStruct((BW, _KC), jnp.float32)
    i_out = jax.ShapeDtypeStruct((BW, _KC), jnp.int32)

    @functools.partial(
        pl.kernel, mesh=mesh,
        out_type=[i_out, f_out, f_out, f_out],
        compiler_params=pltpu.CompilerParams(needs_layout_passes=False),
        scratch_types=[
            pltpu.VMEM((WPW, _FP), jnp.float32),
            pltpu.VMEM((WPW, _FP), jnp.int32),
            pltpu.VMEM((LN,), jnp.float32),
            pltpu.VMEM((D,), jnp.float32),
            pltpu.VMEM((WPW, 96), jnp.int32),
            pltpu.VMEM((WPW, 96), jnp.float32),
            pltpu.VMEM((16,), jnp.float32),
            pltpu.VMEM((16,), jnp.float32),
            pltpu.VMEM((WPW, _KC), jnp.int32),
            pltpu.VMEM((WPW, _KC), jnp.float32),
            pltpu.VMEM((WPW, _KC), jnp.float32),
            pltpu.VMEM((WPW, _KC), jnp.float32),
            pltpu.SemaphoreType.DMA,
        ],
    )
    def sc_kernel(v_h, flat_h, x_h, lut_h, grid_h, step_h,
                  sel_o, val_o, fr_o, dr_o,
                  v_s, flat_s, lut_s, grid_s, idx_s, gat_s, bk_s, step_s,
                  sel_s, val_s, fr_s, dr_s, sem):
        wid = lax.axis_index("s") * 2 + lax.axis_index("c")
        base = wid * WPW
        rows = pl.ds(base, WPW)
        pltpu.sync_copy(v_h.at[rows, :], v_s)
        pltpu.sync_copy(flat_h.at[rows, :], flat_s)
        pltpu.sync_copy(lut_h, lut_s)
        pltpu.sync_copy(grid_h, grid_s)
        pltpu.sync_copy(step_h, step_s)

        zeros16 = jnp.zeros((16,), jnp.int32)
        step = step_s[...]

        def window_body(wl, carry):
            init_k = jnp.full((16,), -2.0, jnp.float32)
            init_i = zeros16

            def merge_body(r, kc):
                bk, bi = kc
                kv = v_s[wl, pl.ds(r * 16, 16)]
                fv = flat_s[wl, pl.ds(r * 16, 16)]
                kv2, fv2 = plsc.sort_key_val(kv, fv, descending=True)
                rk = lax.rev(bk, (0,))
                ri = lax.rev(bi, (0,))
                m = kv2 >= rk
                nk = jnp.where(m, kv2, rk)
                ni = jnp.where(m, fv2, ri)
                sk, si = plsc.sort_key_val(nk, ni, descending=True)
                return (sk, si)

            best_k, _ = lax.fori_loop(
                0, NV, merge_body, (init_k, init_i))

            bk_s[...] = best_k
            lane = lax.iota(jnp.int32, 16)
            big = jnp.full((16,), D * Fk, jnp.int32)

            def assign_body(j, kc):
                bi, prev_v, prev_f = kc
                jv = zeros16 + j
                bkj = plsc.load_gather(bk_s, [jv])
                thresh = jnp.where(bkj == prev_v, prev_f, zeros16 - 1)

                def scan_body(r, acc):
                    kv = v_s[wl, pl.ds(r * 16, 16)]
                    fv = flat_s[wl, pl.ds(r * 16, 16)]
                    hit = (kv == bkj) & (fv > thresh)
                    return jnp.minimum(acc, jnp.where(hit, fv, big))

                part = lax.fori_loop(0, NV, scan_body, big)
                mj = jnp.min(part, axis=0)
                mjs = zeros16 + mj
                bi = jnp.where(lane == jv, mjs, bi)
                return (bi, bkj, mjs)

            best_i, _, _ = lax.fori_loop(
                0, _KC, assign_body,
                (zeros16, jnp.full((16,), -3.0, jnp.float32), zeros16 - 1))

            f16 = best_i % Fk
            d16 = best_i // Fk
            fi16 = jnp.clip(f16, 1, Fk - 2)
            di16 = jnp.clip(d16, 1, D - 2)
            xbase = (base + wl) * plane
            rowb = xbase + d16 * Fk
            idx_s[wl, pl.ds(0, 16)] = rowb + (fi16 - 1)
            idx_s[wl, pl.ds(16, 16)] = rowb + fi16
            idx_s[wl, pl.ds(32, 16)] = rowb + (fi16 + 1)
            colb = xbase + f16
            idx_s[wl, pl.ds(48, 16)] = colb + (di16 - 1) * Fk
            idx_s[wl, pl.ds(64, 16)] = colb + di16 * Fk
            idx_s[wl, pl.ds(80, 16)] = colb + (di16 + 1) * Fk
            pltpu.async_copy(x_h.at[idx_s.at[wl]], gat_s.at[wl], sem)

            sel_s[wl, :] = best_i
            val_s[wl, :] = best_k
            return carry

        lax.fori_loop(0, WPW, window_body, 0)

        def refine_body(wl, carry):
            pltpu.make_async_copy(
                x_h.at[idx_s.at[wl]], gat_s.at[wl], sem).wait()
            best_i = sel_s[wl, pl.ds(0, 16)]
            f16 = best_i % Fk
            d16 = best_i // Fk
            fi16 = jnp.clip(f16, 1, Fk - 2)
            di16 = jnp.clip(d16, 1, D - 2)

            yfm16 = jnp.abs(gat_s[wl, pl.ds(0, 16)])
            yf016 = jnp.abs(gat_s[wl, pl.ds(16, 16)])
            yfp16 = jnp.abs(gat_s[wl, pl.ds(32, 16)])
            ydm16 = jnp.abs(gat_s[wl, pl.ds(48, 16)])
            yd016 = jnp.abs(gat_s[wl, pl.ds(64, 16)])
            ydp16 = jnp.abs(gat_s[wl, pl.ds(80, 16)])

            f_denom = yfm16 - 2.0 * yf016 + yfp16
            f_bad = jnp.abs(f_denom) < 1e-12
            f_safe = jnp.where(f_bad, 1.0, f_denom)
            f_delta = jnp.where(f_bad, 0.0, 0.5 * (yfm16 - yfp16) / f_safe)
            f_delta = jnp.clip(f_delta, -0.5, 0.5)
            sign = jnp.sign(f_delta)
            mag = jnp.abs(f_delta)
            pos = mag / 0.5 * (LN - 1)
            i0 = jnp.clip(pos.astype(jnp.int32), 0, LN - 2)
            frac = pos - i0.astype(jnp.float32)
            l0 = plsc.load_gather(lut_s, [i0])
            l1 = plsc.load_gather(lut_s, [i0 + 1])
            f_delta_c = sign * (l0 * (1.0 - frac) + l1 * frac)
            fr16 = fi16.astype(jnp.float32) + f_delta_c

            d_denom = ydm16 - 2.0 * yd016 + ydp16
            d_bad = jnp.abs(d_denom) < 1e-12
            d_safe = jnp.where(d_bad, 1.0, d_denom)
            d_delta = jnp.where(d_bad, 0.0, 0.5 * (ydm16 - ydp16) / d_safe)
            d_delta = jnp.clip(d_delta, -0.5, 0.5)
            gv = plsc.load_gather(grid_s, [di16])
            dr16 = gv + d_delta * step

            fr_s[wl, :] = fr16
            dr_s[wl, :] = dr16
            return carry

        lax.fori_loop(0, WPW, refine_body, 0)

        pltpu.sync_copy(sel_s, sel_o.at[rows, :])
        pltpu.sync_copy(val_s, val_o.at[rows, :])
        pltpu.sync_copy(fr_s, fr_o.at[rows, :])
        pltpu.sync_copy(dr_s, dr_o.at[rows, :])

    return sc_kernel


def kernel(X, K, dlnf_grid, radius, para_lut):
    B, W, D, Fk = X.shape
    BW = B * W
    G = 32
    Xr = X.reshape(BW, D, Fk)
    v, flat = _stage1(Xr, G)
    LN = para_lut.shape[0]
    sc = _make_stage2(BW, D, Fk, LN)
    step_arr = jnp.broadcast_to(dlnf_grid[1] - dlnf_grid[0], (16,))
    sel, vals, fr, dr = sc(v, flat, X.reshape(-1), para_lut, dlnf_grid,
                           step_arr)
    offset = (jnp.asarray(K) - 16 + jnp.asarray(radius) - 3).astype(jnp.int32)
    flat2 = sel + offset
    d_idx = flat2 // Fk
    f_idx = flat2 % Fk
    peaks = jnp.stack([d_idx, f_idx], axis=-1).reshape(B, W, _KC, 2)
    return (peaks,
            fr.reshape(B, W, _KC),
            dr.reshape(B, W, _KC),
            vals.reshape(B, W, _KC))


# speedup vs baseline: 4.3845x; 1.0095x over previous
import functools

import jax
import jax.numpy as jnp
from jax import lax
from jax.experimental import pallas as pl
from jax.experimental.pallas import tpu as pltpu
from jax.experimental.pallas import tpu_sc as plsc

_KC = 16
_FP = 528


def _stage1_kernel(x_ref, v_out, flat_out, yfm_out, yf0_out, yfp_out,
                   ydm_out, yd0_out, ydp_out):
    G, D, F = x_ref.shape
    amp = jnp.abs(x_ref[...])

    dpad = jnp.full((G, 3, F), -1.0, jnp.float32)
    xd = jnp.concatenate([dpad, amp, dpad], axis=1)
    m2d = jnp.maximum(xd[:, 0:D + 5, :], xd[:, 1:D + 6, :])
    m4d = jnp.maximum(m2d[:, 0:D + 3, :], m2d[:, 2:D + 5, :])
    rowpool = jnp.maximum(jnp.maximum(m4d[:, 0:D, :], m2d[:, 4:D + 4, :]),
                          xd[:, 6:D + 6, :])
    fpad = jnp.full((G, D, 3), -1.0, jnp.float32)
    xf = jnp.concatenate([fpad, rowpool, fpad], axis=2)
    m2f = jnp.maximum(xf[:, :, 0:F + 5], xf[:, :, 1:F + 6])
    m4f = jnp.maximum(m2f[:, :, 0:F + 3], m2f[:, :, 2:F + 5])
    pooled = jnp.maximum(jnp.maximum(m4f[:, :, 0:F], m2f[:, :, 4:F + 4]),
                         xf[:, :, 6:F + 6])

    colmax = jnp.max(amp, axis=1)
    dio = jax.lax.broadcasted_iota(jnp.int32, (G, D, F), 1)
    ismax = amp == colmax[:, None, :]
    bestd = jnp.min(jnp.where(ismax, dio, D), axis=1)
    E = dio == bestd[:, None, :]
    pr = jnp.max(jnp.where(E, pooled, -1.0), axis=1)
    v = jnp.where(colmax >= pr, colmax, 0.0)

    fio = jax.lax.broadcasted_iota(jnp.int32, (G, F), 1)
    flat = bestd * F + fio

    bestdL = jnp.concatenate([bestd[:, 1:F], bestd[:, F - 1:F]], axis=1)
    bestdR = jnp.concatenate([bestd[:, 0:1], bestd[:, 0:F - 1]], axis=1)
    CL = jnp.sum(jnp.where(dio == bestdL[:, None, :], amp, 0.0), axis=1)
    CR = jnp.sum(jnp.where(dio == bestdR[:, None, :], amp, 0.0), axis=1)

    dio2 = jax.lax.broadcasted_iota(jnp.int32, (G, D), 1)
    e1 = jnp.sum(jnp.where(dio2 == bestd[:, F - 1:F], amp[:, :, F - 3], 0.0),
                 axis=1, keepdims=True)
    e2 = jnp.sum(jnp.where(dio2 == bestd[:, 0:1], amp[:, :, 2], 0.0),
                 axis=1, keepdims=True)

    yfm = jnp.concatenate([colmax[:, 0:1], CL[:, 0:F - 2], e1], axis=1)
    yf0 = jnp.concatenate([CR[:, 1:2], colmax[:, 1:F - 1],
                           CL[:, F - 2:F - 1]], axis=1)
    yfp = jnp.concatenate([e2, CR[:, 2:F], colmax[:, F - 1:F]], axis=1)

    di = jnp.clip(bestd, 1, D - 2)
    ydm = jnp.sum(jnp.where(dio == (di - 1)[:, None, :], amp, 0.0), axis=1)
    yd0 = jnp.sum(jnp.where(dio == di[:, None, :], amp, 0.0), axis=1)
    ydp = jnp.sum(jnp.where(dio == (di + 1)[:, None, :], amp, 0.0), axis=1)

    npad = _FP - F

    def padf(a, val, dt):
        return jnp.concatenate(
            [a, jnp.full((G, npad), val, dt)], axis=1).astype(dt)

    v_out[...] = padf(v, -1.0, jnp.float32)
    flat_out[...] = padf(flat, 0, jnp.int32)
    yfm_out[...] = padf(yfm, 0.0, jnp.float32)
    yf0_out[...] = padf(yf0, 0.0, jnp.float32)
    yfp_out[...] = padf(yfp, 0.0, jnp.float32)
    ydm_out[...] = padf(ydm, 0.0, jnp.float32)
    yd0_out[...] = padf(yd0, 0.0, jnp.float32)
    ydp_out[...] = padf(ydp, 0.0, jnp.float32)


def _stage1(Xr, G):
    BW, D, Fk = Xr.shape
    fspec = pl.BlockSpec((G, _FP), lambda i: (i, 0))
    fshape = jax.ShapeDtypeStruct((BW, _FP), jnp.float32)
    ishape = jax.ShapeDtypeStruct((BW, _FP), jnp.int32)
    return pl.pallas_call(
        _stage1_kernel,
        grid=(BW // G,),
        in_specs=[pl.BlockSpec((G, D, Fk), lambda i: (i, 0, 0))],
        out_specs=[fspec] * 8,
        out_shape=[fshape, ishape] + [fshape] * 6,
    )(Xr)


def _make_stage2(BW, D, Fk, LN):
    NW = 32
    WPW = BW // NW
    NV = _FP // 16
    mesh = plsc.VectorSubcoreMesh(core_axis_name="c", subcore_axis_name="s")
    f_out = jax.ShapeDtypeStruct((BW, _KC), jnp.float32)
    i_out = jax.ShapeDtypeStruct((BW, _KC), jnp.int32)
    slab_f = pltpu.VMEM((WPW, _FP), jnp.float32)

    @functools.partial(
        pl.kernel, mesh=mesh,
        out_type=[i_out, f_out, f_out, f_out],
        compiler_params=pltpu.CompilerParams(needs_layout_passes=False),
        scratch_types=[
            slab_f,
            pltpu.VMEM((WPW, _FP), jnp.int32),
            slab_f, slab_f, slab_f,
            slab_f, slab_f, slab_f,
            pltpu.VMEM((LN,), jnp.float32),
            pltpu.VMEM((D,), jnp.float32),
            pltpu.VMEM((16,), jnp.float32),
            pltpu.VMEM((16,), jnp.float32),
            pltpu.VMEM((WPW, _KC), jnp.int32),
            pltpu.VMEM((WPW, _KC), jnp.float32),
            pltpu.VMEM((WPW, _KC), jnp.float32),
            pltpu.VMEM((WPW, _KC), jnp.float32),
        ],
    )
    def sc_kernel(v_h, flat_h, yfm_h, yf0_h, yfp_h, ydm_h, yd0_h, ydp_h,
                  lut_h, grid_h, step_h,
                  sel_o, val_o, fr_o, dr_o,
                  v_s, flat_s, yfm_s, yf0_s, yfp_s, ydm_s, yd0_s, ydp_s,
                  lut_s, grid_s, bk_s, step_s,
                  sel_s, val_s, fr_s, dr_s):
        wid = lax.axis_index("s") * 2 + lax.axis_index("c")
        base = wid * WPW
        rows = pl.ds(base, WPW)
        pltpu.sync_copy(v_h.at[rows, :], v_s)
        pltpu.sync_copy(flat_h.at[rows, :], flat_s)
        pltpu.sync_copy(yfm_h.at[rows, :], yfm_s)
        pltpu.sync_copy(yf0_h.at[rows, :], yf0_s)
        pltpu.sync_copy(yfp_h.at[rows, :], yfp_s)
        pltpu.sync_copy(ydm_h.at[rows, :], ydm_s)
        pltpu.sync_copy(yd0_h.at[rows, :], yd0_s)
        pltpu.sync_copy(ydp_h.at[rows, :], ydp_s)
        pltpu.sync_copy(lut_h, lut_s)
        pltpu.sync_copy(grid_h, grid_s)
        pltpu.sync_copy(step_h, step_s)

        zeros16 = jnp.zeros((16,), jnp.int32)
        step = step_s[...]
        lane = lax.iota(jnp.int32, 16)
        big = jnp.full((16,), D * Fk, jnp.int32)

        def window_body(wl, carry):
            init_k = jnp.full((16,), -2.0, jnp.float32)
            init_i = zeros16

            def merge_body(r, kc):
                bk, bi = kc
                kv = v_s[wl, pl.ds(r * 16, 16)]
                fv = flat_s[wl, pl.ds(r * 16, 16)]
                kv2, fv2 = plsc.sort_key_val(kv, fv, descending=True)
                rk = lax.rev(bk, (0,))
                ri = lax.rev(bi, (0,))
                m = kv2 >= rk
                nk = jnp.where(m, kv2, rk)
                ni = jnp.where(m, fv2, ri)
                sk, si = plsc.sort_key_val(nk, ni, descending=True)
                return (sk, si)

            best_k, _ = lax.fori_loop(
                0, NV, merge_body, (init_k, init_i))

            bk_s[...] = best_k

            def assign_body(j, kc):
                bi, prev_v, prev_f = kc
                jv = zeros16 + j
                bkj = plsc.load_gather(bk_s, [jv])
                thresh = jnp.where(bkj == prev_v, prev_f, zeros16 - 1)

                def scan_body(r, acc):
                    kv = v_s[wl, pl.ds(r * 16, 16)]
                    fv = flat_s[wl, pl.ds(r * 16, 16)]
                    hit = (kv == bkj) & (fv > thresh)
                    return jnp.minimum(acc, jnp.where(hit, fv, big))

                part = lax.fori_loop(0, NV, scan_body, big)
                mj = jnp.min(part, axis=0)
                mjs = zeros16 + mj
                bi = jnp.where(lane == jv, mjs, bi)
                return (bi, bkj, mjs)

            best_i, _, _ = lax.fori_loop(
                0, _KC, assign_body,
                (zeros16, jnp.full((16,), -3.0, jnp.float32), zeros16 - 1))

            f16 = best_i % Fk
            wlv = zeros16 + wl
            yfm16 = plsc.load_gather(yfm_s, [wlv, f16])
            yf016 = plsc.load_gather(yf0_s, [wlv, f16])
            yfp16 = plsc.load_gather(yfp_s, [wlv, f16])
            ydm16 = plsc.load_gather(ydm_s, [wlv, f16])
            yd016 = plsc.load_gather(yd0_s, [wlv, f16])
            ydp16 = plsc.load_gather(ydp_s, [wlv, f16])

            f_denom = yfm16 - 2.0 * yf016 + yfp16
            f_bad = jnp.abs(f_denom) < 1e-12
            f_safe = jnp.where(f_bad, 1.0, f_denom)
            f_delta = jnp.where(f_bad, 0.0, 0.5 * (yfm16 - yfp16) / f_safe)
            f_delta = jnp.clip(f_delta, -0.5, 0.5)
            sign = jnp.sign(f_delta)
            mag = jnp.abs(f_delta)
            pos = mag / 0.5 * (LN - 1)
            i0 = jnp.clip(pos.astype(jnp.int32), 0, LN - 2)
            frac = pos - i0.astype(jnp.float32)
            l0 = plsc.load_gather(lut_s, [i0])
            l1 = plsc.load_gather(lut_s, [i0 + 1])
            f_delta_c = sign * (l0 * (1.0 - frac) + l1 * frac)
            fi16 = jnp.clip(f16, 1, Fk - 2)
            fr16 = fi16.astype(jnp.float32) + f_delta_c

            d_denom = ydm16 - 2.0 * yd016 + ydp16
            d_bad = jnp.abs(d_denom) < 1e-12
            d_safe = jnp.where(d_bad, 1.0, d_denom)
            d_delta = jnp.where(d_bad, 0.0, 0.5 * (ydm16 - ydp16) / d_safe)
            d_delta = jnp.clip(d_delta, -0.5, 0.5)
            d16 = best_i // Fk
            di16 = jnp.clip(d16, 1, D - 2)
            gv = plsc.load_gather(grid_s, [di16])
            dr16 = gv + d_delta * step

            sel_s[wl, :] = best_i
            val_s[wl, :] = best_k
            fr_s[wl, :] = fr16
            dr_s[wl, :] = dr16
            return carry

        lax.fori_loop(0, WPW, window_body, 0)

        pltpu.sync_copy(sel_s, sel_o.at[rows, :])
        pltpu.sync_copy(val_s, val_o.at[rows, :])
        pltpu.sync_copy(fr_s, fr_o.at[rows, :])
        pltpu.sync_copy(dr_s, dr_o.at[rows, :])

    return sc_kernel


def kernel(X, K, dlnf_grid, radius, para_lut):
    B, W, D, Fk = X.shape
    BW = B * W
    G = 32
    Xr = X.reshape(BW, D, Fk)
    outs1 = _stage1(Xr, G)
    LN = para_lut.shape[0]
    sc = _make_stage2(BW, D, Fk, LN)
    step_arr = jnp.broadcast_to(dlnf_grid[1] - dlnf_grid[0], (16,))
    sel, vals, fr, dr = sc(*outs1, para_lut, dlnf_grid, step_arr)
    offset = (jnp.asarray(K) - 16 + jnp.asarray(radius) - 3).astype(jnp.int32)
    flat2 = sel + offset
    d_idx = flat2 // Fk
    f_idx = flat2 % Fk
    peaks = jnp.stack([d_idx, f_idx], axis=-1).reshape(B, W, _KC, 2)
    return (peaks,
            fr.reshape(B, W, _KC),
            dr.reshape(B, W, _KC),
            vals.reshape(B, W, _KC))
